# Initial kernel scaffold; baseline (speedup 1.0000x reference)
import functools

import jax
import jax.numpy as jnp
from jax import lax
from jax.experimental import pallas as pl
from jax.experimental.pallas import tpu as pltpu
from jax.experimental.pallas import tpu_sc as plsc

N_ROWS = 100000
F = 128
S = 128
L = 16
NC, NS = 2, 16
NW = NC * NS
CHUNK = 200
N_CHUNKS = N_ROWS // CHUNK


def _sc_partials(x, batch):
    q, r = divmod(N_CHUNKS, NW)
    mesh = plsc.VectorSubcoreMesh(
        core_axis_name="c", subcore_axis_name="s",
        num_cores=NC, num_subcores=NS)

    @functools.partial(
        pl.kernel,
        mesh=mesh,
        out_type=[
            jax.ShapeDtypeStruct((NW, S, F), jnp.float32),
            jax.ShapeDtypeStruct((NW, S, F), jnp.float32),
            jax.ShapeDtypeStruct((NW, S, L), jnp.float32),
        ],
        scratch_types=[
            pltpu.VMEM((CHUNK, F), jnp.float32),
            pltpu.VMEM((CHUNK,), jnp.int32),
            pltpu.VMEM((S, F), jnp.float32),
            pltpu.VMEM((S, F), jnp.float32),
            pltpu.VMEM((S, L), jnp.float32),
        ],
    )
    def k(x_hbm, b_hbm, sum_hbm, max_hbm, cnt_hbm, xv, bv, asum, amax, acnt):
        wid = lax.axis_index("s") * NC + lax.axis_index("c")
        lanes = lax.iota(jnp.int32, L)
        zeros = jnp.zeros((L,), jnp.float32)
        ninf = jnp.full((L,), -jnp.inf, jnp.float32)

        def init_body(i2, _):
            for k8 in range(F // L):
                asum[i2, pl.ds(L * k8, L)] = zeros
                amax[i2, pl.ds(L * k8, L)] = ninf
            acnt[i2, pl.ds(0, L)] = zeros
            return 0

        lax.fori_loop(0, S, init_body, 0)

        c0 = wid * q + jnp.minimum(wid, r)
        c1 = c0 + q + (wid < r).astype(jnp.int32)

        def chunk_body(c, carry):
            pltpu.sync_copy(x_hbm.at[pl.ds(c * CHUNK, CHUNK), :], xv)
            pltpu.sync_copy(b_hbm.at[pl.ds(c * CHUNK, CHUNK)], bv)

            def row_body(i, rc):
                prev = rc[0]
                svec = rc[1:9]
                mvec = rc[9:17]
                cnt = rc[17]
                seg = plsc.load_gather(bv, [jnp.full((L,), i, jnp.int32)])
                same = seg == prev
                news, newm = [], []
                for k8 in range(F // L):
                    xk = xv[i, pl.ds(L * k8, L)]
                    sk = jnp.where(same, svec[k8] + xk, xk)
                    mk = jnp.where(same, jnp.maximum(mvec[k8], xk), xk)
                    col = lanes + (L * k8)
                    plsc.store_scatter(asum, [seg, col], sk)
                    plsc.store_scatter(amax, [seg, col], mk)
                    news.append(sk)
                    newm.append(mk)
                newc = jnp.where(same, cnt + 1.0, jnp.ones((L,), jnp.float32))
                plsc.store_scatter(acnt, [seg, lanes], newc)
                return (seg, *news, *newm, newc)

            return lax.fori_loop(0, CHUNK, row_body, carry)

        init_carry = (jnp.full((L,), -1, jnp.int32),) \
            + (zeros,) * 8 + (ninf,) * 8 + (zeros,)
        lax.fori_loop(c0, c1, chunk_body, init_carry)

        pltpu.sync_copy(asum, sum_hbm.at[wid])
        pltpu.sync_copy(amax, max_hbm.at[wid])
        pltpu.sync_copy(acnt, cnt_hbm.at[wid])

    return k(x, batch)


def _tc_merge(sum_p, max_p, cnt_p):
    def body(s_ref, m_ref, c_ref, o_ref):
        s = jnp.sum(s_ref[...], axis=0)
        m = jnp.max(m_ref[...], axis=0)
        c = jnp.sum(c_ref[...], axis=0)
        cn = c[:, 0:1]
        o_ref[:, 0:F] = s / jnp.maximum(cn, 1.0)
        o_ref[:, F:2 * F] = m

    return pl.pallas_call(
        body,
        out_shape=jax.ShapeDtypeStruct((S, 2 * F), jnp.float32),
    )(sum_p, max_p, cnt_p)


def kernel(x, batch):
    sum_p, max_p, cnt_p = _sc_partials(x, batch.astype(jnp.int32))
    return _tc_merge(sum_p, max_p, cnt_p)


# speedup vs baseline: 3.4357x; 3.4357x over previous
import functools

import jax
import jax.numpy as jnp
from jax import lax
from jax.experimental import pallas as pl
from jax.experimental.pallas import tpu as pltpu
from jax.experimental.pallas import tpu_sc as plsc

N_ROWS = 100000
F = 128
S = 128
L = 16
NC, NS = 2, 16
NW = NC * NS
CHUNK = 200
N_CHUNKS = N_ROWS // CHUNK


def _sc_partials(x, batch):
    q, r = divmod(N_CHUNKS, NW)
    mesh = plsc.VectorSubcoreMesh(
        core_axis_name="c", subcore_axis_name="s",
        num_cores=NC, num_subcores=NS)

    @functools.partial(
        pl.kernel,
        mesh=mesh,
        compiler_params=pltpu.CompilerParams(needs_layout_passes=False),
        out_type=[
            jax.ShapeDtypeStruct((NW, S, F), jnp.float32),
            jax.ShapeDtypeStruct((NW, S, F), jnp.float32),
            jax.ShapeDtypeStruct((NW, S, L), jnp.float32),
        ],
        scratch_types=[
            pltpu.VMEM((CHUNK, F), jnp.float32),
            pltpu.VMEM((CHUNK,), jnp.int32),
            pltpu.VMEM((S, F), jnp.float32),
            pltpu.VMEM((S, F), jnp.float32),
            pltpu.VMEM((S, L), jnp.float32),
        ],
    )
    def k(x_hbm, b_hbm, sum_hbm, max_hbm, cnt_hbm, xv, bv, asum, amax, acnt):
        wid = lax.axis_index("s") * NC + lax.axis_index("c")
        lanes = lax.iota(jnp.int32, L)
        zeros = jnp.zeros((L,), jnp.float32)
        ninf = jnp.full((L,), -jnp.inf, jnp.float32)

        def init_body(i2, _):
            for k8 in range(F // L):
                asum[i2, pl.ds(L * k8, L)] = zeros
                amax[i2, pl.ds(L * k8, L)] = ninf
            acnt[i2, pl.ds(0, L)] = zeros
            return 0

        lax.fori_loop(0, S, init_body, 0)

        c0 = wid * q + jnp.minimum(wid, r)
        c1 = c0 + q + (wid < r).astype(jnp.int32)

        def chunk_body(c, carry):
            pltpu.sync_copy(x_hbm.at[pl.ds(c * CHUNK, CHUNK), :], xv)
            pltpu.sync_copy(b_hbm.at[pl.ds(c * CHUNK, CHUNK)], bv)

            def row_body(i, rc):
                prev = rc[0]
                svec = rc[1:9]
                mvec = rc[9:17]
                cnt = rc[17]
                seg = plsc.load_gather(bv, [jnp.full((L,), i, jnp.int32)])
                same = seg == prev
                news, newm = [], []
                for k8 in range(F // L):
                    xk = xv[i, pl.ds(L * k8, L)]
                    sk = jnp.where(same, svec[k8] + xk, xk)
                    mk = jnp.where(same, jnp.maximum(mvec[k8], xk), xk)
                    col = lanes + (L * k8)
                    plsc.store_scatter(asum, [seg, col], sk)
                    plsc.store_scatter(amax, [seg, col], mk)
                    news.append(sk)
                    newm.append(mk)
                newc = jnp.where(same, cnt + 1.0, jnp.ones((L,), jnp.float32))
                plsc.store_scatter(acnt, [seg, lanes], newc)
                return (seg, *news, *newm, newc)

            return lax.fori_loop(0, CHUNK, row_body, carry)

        init_carry = (jnp.full((L,), -1, jnp.int32),) \
            + (zeros,) * 8 + (ninf,) * 8 + (zeros,)
        lax.fori_loop(c0, c1, chunk_body, init_carry)

        pltpu.sync_copy(asum, sum_hbm.at[wid])
        pltpu.sync_copy(amax, max_hbm.at[wid])
        pltpu.sync_copy(acnt, cnt_hbm.at[wid])

    return k(x, batch)


def _tc_merge(sum_p, max_p, cnt_p):
    def body(s_ref, m_ref, c_ref, o_ref):
        s = jnp.sum(s_ref[...], axis=0)
        m = jnp.max(m_ref[...], axis=0)
        c = jnp.sum(c_ref[...], axis=0)
        cn = c[:, 0:1]
        o_ref[:, 0:F] = s / jnp.maximum(cn, 1.0)
        o_ref[:, F:2 * F] = m

    return pl.pallas_call(
        body,
        out_shape=jax.ShapeDtypeStruct((S, 2 * F), jnp.float32),
    )(sum_p, max_p, cnt_p)


def kernel(x, batch):
    sum_p, max_p, cnt_p = _sc_partials(x, batch.astype(jnp.int32))
    return _tc_merge(sum_p, max_p, cnt_p)


# speedup vs baseline: 8.2027x; 2.3875x over previous
import functools

import jax
import jax.numpy as jnp
from jax import lax
from jax.experimental import pallas as pl
from jax.experimental.pallas import tpu as pltpu
from jax.experimental.pallas import tpu_sc as plsc

N_ROWS = 100000
F = 128
S = 128
L = 16
NC, NS = 2, 16
NW = NC * NS
CHUNK = 400
N_CHUNKS = N_ROWS // CHUNK
GROUPS = CHUNK // L


def _sc_partials(x, batch):
    q, r = divmod(N_CHUNKS, NW)
    mesh = plsc.VectorSubcoreMesh(
        core_axis_name="c", subcore_axis_name="s",
        num_cores=NC, num_subcores=NS)

    @functools.partial(
        pl.kernel,
        mesh=mesh,
        compiler_params=pltpu.CompilerParams(needs_layout_passes=False),
        out_type=[
            jax.ShapeDtypeStruct((NW, S, F), jnp.float32),
            jax.ShapeDtypeStruct((NW, S, F), jnp.float32),
            jax.ShapeDtypeStruct((NW, S, L), jnp.float32),
        ],
        scratch_types=[
            pltpu.VMEM((CHUNK, F), jnp.float32),
            pltpu.VMEM((CHUNK,), jnp.int32),
            pltpu.VMEM((S, F), jnp.float32),
            pltpu.VMEM((S, F), jnp.float32),
            pltpu.VMEM((S, L), jnp.float32),
        ],
    )
    def k(x_hbm, b_hbm, sum_hbm, max_hbm, cnt_hbm, xv, bv, asum, amax, acnt):
        wid = lax.axis_index("s") * NC + lax.axis_index("c")
        lanes = lax.iota(jnp.int32, L)
        zeros = jnp.zeros((L,), jnp.float32)
        ninf = jnp.full((L,), -jnp.inf, jnp.float32)

        def init_body(i2, _):
            for k8 in range(F // L):
                asum[i2, pl.ds(L * k8, L)] = zeros
                amax[i2, pl.ds(L * k8, L)] = ninf
            acnt[i2, pl.ds(0, L)] = zeros
            return 0

        lax.fori_loop(0, S, init_body, 0)

        c0 = wid * q + jnp.minimum(wid, r)
        c1 = c0 + q + (wid < r).astype(jnp.int32)

        def row_body(i, rc):
            prev = rc[0]
            svec = rc[1:9]
            mvec = rc[9:17]
            cnt = rc[17]
            seg = plsc.load_gather(bv, [jnp.full((L,), i, jnp.int32)])
            same = seg == prev
            news, newm = [], []
            for k8 in range(F // L):
                xk = xv[i, pl.ds(L * k8, L)]
                sk = jnp.where(same, svec[k8] + xk, xk)
                mk = jnp.where(same, jnp.maximum(mvec[k8], xk), xk)
                col = lanes + (L * k8)
                plsc.store_scatter(asum, [seg, col], sk)
                plsc.store_scatter(amax, [seg, col], mk)
                news.append(sk)
                newm.append(mk)
            newc = jnp.where(same, cnt + 1.0, jnp.ones((L,), jnp.float32))
            plsc.store_scatter(acnt, [seg, lanes], newc)
            return (seg, *news, *newm, newc)

        def chunk_body(c, carry):
            pltpu.sync_copy(x_hbm.at[pl.ds(c * CHUNK, CHUNK), :], xv)
            pltpu.sync_copy(b_hbm.at[pl.ds(c * CHUNK, CHUNK)], bv)

            def group_body(g, gc):
                base = g * L
                bvec = bv[pl.ds(base, L)]
                prev = gc[0]
                fast = jnp.all(bvec == prev)

                def fast_fn(rc):
                    svec = rc[1:9]
                    mvec = rc[9:17]
                    cnt = rc[17]
                    news, newm = [], []
                    for k8 in range(F // L):
                        xs = [xv[base + j, pl.ds(L * k8, L)]
                              for j in range(L)]
                        ms = xs
                        while len(xs) > 1:
                            xs = [xs[2 * t] + xs[2 * t + 1]
                                  for t in range(len(xs) // 2)]
                        while len(ms) > 1:
                            ms = [jnp.maximum(ms[2 * t], ms[2 * t + 1])
                                  for t in range(len(ms) // 2)]
                        sk = svec[k8] + xs[0]
                        mk = jnp.maximum(mvec[k8], ms[0])
                        col = lanes + (L * k8)
                        plsc.store_scatter(asum, [bvec, col], sk)
                        plsc.store_scatter(amax, [bvec, col], mk)
                        news.append(sk)
                        newm.append(mk)
                    newc = cnt + jnp.float32(L)
                    plsc.store_scatter(acnt, [bvec, lanes], newc)
                    return (rc[0], *news, *newm, newc)

                def slow_fn(rc):
                    return lax.fori_loop(base, base + L, row_body, rc)

                return lax.cond(fast, fast_fn, slow_fn, gc)

            return lax.fori_loop(0, GROUPS, group_body, carry)

        init_carry = (jnp.full((L,), -1, jnp.int32),) \
            + (zeros,) * 8 + (ninf,) * 8 + (zeros,)
        lax.fori_loop(c0, c1, chunk_body, init_carry)

        pltpu.sync_copy(asum, sum_hbm.at[wid])
        pltpu.sync_copy(amax, max_hbm.at[wid])
        pltpu.sync_copy(acnt, cnt_hbm.at[wid])

    return k(x, batch)


def _tc_merge(sum_p, max_p, cnt_p):
    def body(s_ref, m_ref, c_ref, o_ref):
        s = jnp.sum(s_ref[...], axis=0)
        m = jnp.max(m_ref[...], axis=0)
        c = jnp.sum(c_ref[...], axis=0)
        cn = c[:, 0:1]
        o_ref[:, 0:F] = s / jnp.maximum(cn, 1.0)
        o_ref[:, F:2 * F] = m

    return pl.pallas_call(
        body,
        out_shape=jax.ShapeDtypeStruct((S, 2 * F), jnp.float32),
    )(sum_p, max_p, cnt_p)


def kernel(x, batch):
    sum_p, max_p, cnt_p = _sc_partials(x, batch.astype(jnp.int32))
    return _tc_merge(sum_p, max_p, cnt_p)


# speedup vs baseline: 10.8572x; 1.3236x over previous
import functools

import jax
import jax.numpy as jnp
from jax import lax
from jax.experimental import pallas as pl
from jax.experimental.pallas import tpu as pltpu
from jax.experimental.pallas import tpu_sc as plsc

N_ROWS = 100000
F = 128
S = 128
L = 16
NC, NS = 2, 16
NW = NC * NS
CHUNK = 160
N_CHUNKS = N_ROWS // CHUNK
GROUPS = CHUNK // L


def _sc_partials(x, batch):
    q, r = divmod(N_CHUNKS, NW)
    mesh = plsc.VectorSubcoreMesh(
        core_axis_name="c", subcore_axis_name="s",
        num_cores=NC, num_subcores=NS)

    @functools.partial(
        pl.kernel,
        mesh=mesh,
        compiler_params=pltpu.CompilerParams(needs_layout_passes=False),
        out_type=[
            jax.ShapeDtypeStruct((NW, S, F), jnp.float32),
            jax.ShapeDtypeStruct((NW, S, F), jnp.float32),
            jax.ShapeDtypeStruct((NW, S, L), jnp.float32),
        ],
        scratch_types=[
            pltpu.VMEM((CHUNK, F), jnp.float32),
            pltpu.VMEM((CHUNK, F), jnp.float32),
            pltpu.VMEM((CHUNK,), jnp.int32),
            pltpu.VMEM((CHUNK,), jnp.int32),
            pltpu.VMEM((S, F), jnp.float32),
            pltpu.VMEM((S, F), jnp.float32),
            pltpu.VMEM((S, L), jnp.float32),
            pltpu.SemaphoreType.DMA,
            pltpu.SemaphoreType.DMA,
            pltpu.SemaphoreType.DMA,
            pltpu.SemaphoreType.DMA,
        ],
    )
    def k(x_hbm, b_hbm, sum_hbm, max_hbm, cnt_hbm,
          xv0, xv1, bv0, bv1, asum, amax, acnt,
          semx0, semx1, semb0, semb1):
        wid = lax.axis_index("s") * NC + lax.axis_index("c")
        lanes = lax.iota(jnp.int32, L)
        zeros = jnp.zeros((L,), jnp.float32)
        ninf = jnp.full((L,), -jnp.inf, jnp.float32)

        def init_body(i2, _):
            for k8 in range(F // L):
                asum[i2, pl.ds(L * k8, L)] = zeros
                amax[i2, pl.ds(L * k8, L)] = ninf
            acnt[i2, pl.ds(0, L)] = zeros
            return 0

        lax.fori_loop(0, S, init_body, 0)

        c0 = wid * q + jnp.minimum(wid, r)
        c1 = c0 + q + (wid < r).astype(jnp.int32)

        def dma_x(c, xv, semx):
            return pltpu.make_async_copy(
                x_hbm.at[pl.ds(c * CHUNK, CHUNK), :], xv, semx)

        def dma_b(c, bv, semb):
            return pltpu.make_async_copy(
                b_hbm.at[pl.ds(c * CHUNK, CHUNK)], bv, semb)

        def start(c, xv, bv, semx, semb):
            dma_x(c, xv, semx).start()
            dma_b(c, bv, semb).start()

        def wait(c, xv, bv, semx, semb):
            dma_x(c, xv, semx).wait()
            dma_b(c, bv, semb).wait()

        def make_row_body(xv, bv):
            def row_body(i, rc):
                prev = rc[0]
                svec = rc[1:9]
                mvec = rc[9:17]
                cnt = rc[17]
                seg = plsc.load_gather(bv, [jnp.full((L,), i, jnp.int32)])
                same = seg == prev
                news, newm = [], []
                for k8 in range(F // L):
                    xk = xv[i, pl.ds(L * k8, L)]
                    sk = jnp.where(same, svec[k8] + xk, xk)
                    mk = jnp.where(same, jnp.maximum(mvec[k8], xk), xk)
                    col = lanes + (L * k8)
                    plsc.store_scatter(asum, [seg, col], sk)
                    plsc.store_scatter(amax, [seg, col], mk)
                    news.append(sk)
                    newm.append(mk)
                newc = jnp.where(same, cnt + 1.0, jnp.ones((L,), jnp.float32))
                plsc.store_scatter(acnt, [seg, lanes], newc)
                return (seg, *news, *newm, newc)
            return row_body

        def process(xv, bv, carry):
            row_body = make_row_body(xv, bv)

            def group_body(g, gc):
                base = g * L
                bvec = bv[pl.ds(base, L)]
                prev = gc[0]
                fast = jnp.all(bvec == prev)

                def fast_fn(rc):
                    svec = rc[1:9]
                    mvec = rc[9:17]
                    cnt = rc[17]
                    news, newm = [], []
                    for k8 in range(F // L):
                        xs = [xv[base + j, pl.ds(L * k8, L)]
                              for j in range(L)]
                        ms = xs
                        while len(xs) > 1:
                            xs = [xs[2 * t] + xs[2 * t + 1]
                                  for t in range(len(xs) // 2)]
                        while len(ms) > 1:
                            ms = [jnp.maximum(ms[2 * t], ms[2 * t + 1])
                                  for t in range(len(ms) // 2)]
                        sk = svec[k8] + xs[0]
                        mk = jnp.maximum(mvec[k8], ms[0])
                        col = lanes + (L * k8)
                        plsc.store_scatter(asum, [bvec, col], sk)
                        plsc.store_scatter(amax, [bvec, col], mk)
                        news.append(sk)
                        newm.append(mk)
                    newc = cnt + jnp.float32(L)
                    plsc.store_scatter(acnt, [bvec, lanes], newc)
                    return (rc[0], *news, *newm, newc)

                def slow_fn(rc):
                    return lax.fori_loop(base, base + L, row_body, rc)

                return lax.cond(fast, fast_fn, slow_fn, gc)

            return lax.fori_loop(0, GROUPS, group_body, carry)

        start(c0, xv0, bv0, semx0, semb0)

        def pair_body(p, carry):
            ce = c0 + 2 * p
            co = ce + 1

            @pl.when(co < c1)
            def _():
                start(co, xv1, bv1, semx1, semb1)

            wait(ce, xv0, bv0, semx0, semb0)
            carry = process(xv0, bv0, carry)

            @pl.when(ce + 2 < c1)
            def _():
                start(ce + 2, xv0, bv0, semx0, semb0)

            def odd_fn(rc):
                wait(co, xv1, bv1, semx1, semb1)
                return process(xv1, bv1, rc)

            return lax.cond(co < c1, odd_fn, lambda rc: rc, carry)

        init_carry = (jnp.full((L,), -1, jnp.int32),) \
            + (zeros,) * 8 + (ninf,) * 8 + (zeros,)
        pairs = (c1 - c0 + 1) // 2
        lax.fori_loop(0, pairs, pair_body, init_carry)

        pltpu.sync_copy(asum, sum_hbm.at[wid])
        pltpu.sync_copy(amax, max_hbm.at[wid])
        pltpu.sync_copy(acnt, cnt_hbm.at[wid])

    return k(x, batch)


def _tc_merge(sum_p, max_p, cnt_p):
    def body(s_ref, m_ref, c_ref, o_ref):
        s = jnp.sum(s_ref[...], axis=0)
        m = jnp.max(m_ref[...], axis=0)
        c = jnp.sum(c_ref[...], axis=0)
        cn = c[:, 0:1]
        o_ref[:, 0:F] = s / jnp.maximum(cn, 1.0)
        o_ref[:, F:2 * F] = m

    return pl.pallas_call(
        body,
        out_shape=jax.ShapeDtypeStruct((S, 2 * F), jnp.float32),
    )(sum_p, max_p, cnt_p)


def kernel(x, batch):
    sum_p, max_p, cnt_p = _sc_partials(x, batch.astype(jnp.int32))
    return _tc_merge(sum_p, max_p, cnt_p)


# speedup vs baseline: 12.5158x; 1.1528x over previous
import functools

import jax
import jax.numpy as jnp
from jax import lax
from jax.experimental import pallas as pl
from jax.experimental.pallas import tpu as pltpu
from jax.experimental.pallas import tpu_sc as plsc

N_ROWS = 100000
F = 128
S = 128
L = 16
NC, NS = 2, 16
NW = NC * NS
CHUNK = 160
N_CHUNKS = N_ROWS // CHUNK
GROUPS = CHUNK // L


def _sc_partials(x, batch):
    q, r = divmod(N_CHUNKS, NW)
    mesh = plsc.VectorSubcoreMesh(
        core_axis_name="c", subcore_axis_name="s",
        num_cores=NC, num_subcores=NS)

    @functools.partial(
        pl.kernel,
        mesh=mesh,
        compiler_params=pltpu.CompilerParams(needs_layout_passes=False),
        out_type=[
            jax.ShapeDtypeStruct((NW, S, F), jnp.float32),
            jax.ShapeDtypeStruct((NW, S, F), jnp.float32),
            jax.ShapeDtypeStruct((NW, S, L), jnp.float32),
        ],
        scratch_types=[
            pltpu.VMEM((CHUNK, F), jnp.float32),
            pltpu.VMEM((CHUNK, F), jnp.float32),
            pltpu.VMEM((CHUNK,), jnp.int32),
            pltpu.VMEM((CHUNK,), jnp.int32),
            pltpu.VMEM((S, F), jnp.float32),
            pltpu.VMEM((S, F), jnp.float32),
            pltpu.VMEM((S, L), jnp.float32),
            pltpu.SemaphoreType.DMA,
            pltpu.SemaphoreType.DMA,
            pltpu.SemaphoreType.DMA,
            pltpu.SemaphoreType.DMA,
        ],
    )
    def k(x_hbm, b_hbm, sum_hbm, max_hbm, cnt_hbm,
          xv0, xv1, bv0, bv1, asum, amax, acnt,
          semx0, semx1, semb0, semb1):
        wid = lax.axis_index("s") * NC + lax.axis_index("c")
        lanes = lax.iota(jnp.int32, L)
        zeros = jnp.zeros((L,), jnp.float32)
        ninf = jnp.full((L,), -jnp.inf, jnp.float32)

        c0 = wid * q + jnp.minimum(wid, r)
        c1 = c0 + q + (wid < r).astype(jnp.int32)

        def dma_x(c, xv, semx):
            return pltpu.make_async_copy(
                x_hbm.at[pl.ds(c * CHUNK, CHUNK), :], xv, semx)

        def dma_b(c, bv, semb):
            return pltpu.make_async_copy(
                b_hbm.at[pl.ds(c * CHUNK, CHUNK)], bv, semb)

        def start(c, xv, bv, semx, semb):
            dma_x(c, xv, semx).start()
            dma_b(c, bv, semb).start()

        def wait(c, xv, bv, semx, semb):
            dma_x(c, xv, semx).wait()
            dma_b(c, bv, semb).wait()

        start(c0, xv0, bv0, semx0, semb0)

        def init_body(i2, _):
            for k8 in range(F // L):
                asum[i2, pl.ds(L * k8, L)] = zeros
                amax[i2, pl.ds(L * k8, L)] = ninf
            acnt[i2, pl.ds(0, L)] = zeros
            return 0

        lax.fori_loop(0, S, init_body, 0)

        def make_row_body(xv, bv):
            def row_body(i, rc):
                prev = rc[0]
                svec = rc[1:9]
                mvec = rc[9:17]
                cnt = rc[17]
                seg = plsc.load_gather(bv, [jnp.full((L,), i, jnp.int32)])
                same = seg == prev
                news, newm = [], []
                for k8 in range(F // L):
                    xk = xv[i, pl.ds(L * k8, L)]
                    sk = jnp.where(same, svec[k8] + xk, xk)
                    mk = jnp.where(same, jnp.maximum(mvec[k8], xk), xk)
                    col = lanes + (L * k8)
                    plsc.store_scatter(asum, [seg, col], sk)
                    plsc.store_scatter(amax, [seg, col], mk)
                    news.append(sk)
                    newm.append(mk)
                newc = jnp.where(same, cnt + 1.0, jnp.ones((L,), jnp.float32))
                plsc.store_scatter(acnt, [seg, lanes], newc)
                return (seg, *news, *newm, newc)
            return row_body

        def process(xv, bv, carry):
            row_body = make_row_body(xv, bv)

            def group_body(g, gc):
                base = g * L
                bvec = bv[pl.ds(base, L)]
                prev = gc[0]
                fast = jnp.all(bvec == prev)

                def fast_fn(rc):
                    svec = rc[1:9]
                    mvec = rc[9:17]
                    cnt = rc[17]
                    news, newm = [], []
                    for k8 in range(F // L):
                        xs = [xv[base + j, pl.ds(L * k8, L)]
                              for j in range(L)]
                        ms = xs
                        while len(xs) > 1:
                            xs = [xs[2 * t] + xs[2 * t + 1]
                                  for t in range(len(xs) // 2)]
                        while len(ms) > 1:
                            ms = [jnp.maximum(ms[2 * t], ms[2 * t + 1])
                                  for t in range(len(ms) // 2)]
                        news.append(svec[k8] + xs[0])
                        newm.append(jnp.maximum(mvec[k8], ms[0]))
                    for k8 in range(F // L):
                        col = lanes + (L * k8)
                        plsc.store_scatter(asum, [bvec, col], news[k8])
                        plsc.store_scatter(amax, [bvec, col], newm[k8])
                    newc = cnt + jnp.float32(L)
                    plsc.store_scatter(acnt, [bvec, lanes], newc)
                    return (rc[0], *news, *newm, newc)

                def slow_fn(rc):
                    return lax.fori_loop(base, base + L, row_body, rc)

                return lax.cond(fast, fast_fn, slow_fn, gc)

            return lax.fori_loop(0, GROUPS, group_body, carry)

        def pair_body(p, carry):
            ce = c0 + 2 * p
            co = ce + 1

            @pl.when(co < c1)
            def _():
                start(co, xv1, bv1, semx1, semb1)

            wait(ce, xv0, bv0, semx0, semb0)
            carry = process(xv0, bv0, carry)

            @pl.when(ce + 2 < c1)
            def _():
                start(ce + 2, xv0, bv0, semx0, semb0)

            def odd_fn(rc):
                wait(co, xv1, bv1, semx1, semb1)
                return process(xv1, bv1, rc)

            return lax.cond(co < c1, odd_fn, lambda rc: rc, carry)

        init_carry = (jnp.full((L,), -1, jnp.int32),) \
            + (zeros,) * 8 + (ninf,) * 8 + (zeros,)
        pairs = (c1 - c0 + 1) // 2
        lax.fori_loop(0, pairs, pair_body, init_carry)

        pltpu.sync_copy(asum, sum_hbm.at[wid])
        pltpu.sync_copy(amax, max_hbm.at[wid])
        pltpu.sync_copy(acnt, cnt_hbm.at[wid])

    return k(x, batch)


def _tc_merge(sum_p, max_p, cnt_p):
    def body(s_ref, m_ref, c_ref, o_ref):
        s = jnp.sum(s_ref[...], axis=0)
        m = jnp.max(m_ref[...], axis=0)
        c = jnp.sum(c_ref[...], axis=0)
        cn = c[:, 0:1]
        o_ref[:, 0:F] = s / jnp.maximum(cn, 1.0)
        o_ref[:, F:2 * F] = m

    return pl.pallas_call(
        body,
        out_shape=jax.ShapeDtypeStruct((S, 2 * F), jnp.float32),
    )(sum_p, max_p, cnt_p)


def kernel(x, batch):
    sum_p, max_p, cnt_p = _sc_partials(x, batch.astype(jnp.int32))
    return _tc_merge(sum_p, max_p, cnt_p)
